# baseline (speedup 1.0000x reference)
import functools

import jax
import jax.numpy as jnp
from jax import lax
from jax.experimental import pallas as pl
from jax.experimental.pallas import tpu as pltpu
from jax.experimental.pallas import tpu_sc as plsc

_NC = 2
_NS = 16
_NW = _NC * _NS

_CHUNK = 128


def _gather_body(n_chunks, D, row0, u_idx, i_idx, utab, itab, out,
                 uidx_v, iidx_v, ubuf0, ubuf1, ibuf0, ibuf1,
                 ugs0, ugs1, igs0, igs1, uss0, uss1, iss0, iss1):
    wid = lax.axis_index("s") * _NC + lax.axis_index("c")
    base = wid * (n_chunks * _CHUNK)
    ubuf, ibuf = (ubuf0, ubuf1), (ibuf0, ibuf1)
    ugs, igs = (ugs0, ugs1), (igs0, igs1)
    uss, iss = (uss0, uss1), (iss0, iss1)
    pltpu.sync_copy(u_idx.at[pl.ds(row0 + wid * n_chunks, n_chunks)], uidx_v)
    pltpu.sync_copy(i_idx.at[pl.ds(row0 + wid * n_chunks, n_chunks)], iidx_v)
    ug = [None] * n_chunks
    ig = [None] * n_chunks
    ust = [None] * n_chunks
    ist = [None] * n_chunks
    for c in range(min(2, n_chunks)):
        ug[c] = pltpu.async_copy(utab.at[uidx_v.at[c]], ubuf[c % 2], ugs[c % 2])
        ig[c] = pltpu.async_copy(itab.at[iidx_v.at[c]], ibuf[c % 2], igs[c % 2])
    for c in range(n_chunks):
        s = c % 2
        off = base + c * _CHUNK
        ug[c].wait()
        ust[c] = pltpu.async_copy(
            ubuf[s], out.at[pl.ds(off, _CHUNK), pl.ds(0, D)], uss[s])
        ig[c].wait()
        ist[c] = pltpu.async_copy(
            ibuf[s], out.at[pl.ds(off, _CHUNK), pl.ds(D, D)], iss[s])
        if c + 2 < n_chunks:
            ust[c].wait()
            ug[c + 2] = pltpu.async_copy(utab.at[uidx_v.at[c + 2]], ubuf[s], ugs[s])
            ist[c].wait()
            ig[c + 2] = pltpu.async_copy(itab.at[iidx_v.at[c + 2]], ibuf[s], igs[s])
    for c in range(max(0, n_chunks - 2), n_chunks):
        ust[c].wait()
        ist[c].wait()


@functools.partial(jax.jit, static_argnums=(4, 5))
def _gather(user, item, user_table, item_table, chunk, n_pipe):
    B = user.shape[0] // n_pipe
    D = user_table.shape[1]
    assert B % (_NW * _CHUNK) == 0
    n_chunks = B // (_NW * _CHUNK)
    row0 = chunk * (B // _CHUNK)
    mesh = plsc.VectorSubcoreMesh(core_axis_name="c", subcore_axis_name="s")
    k = pl.kernel(
        functools.partial(_gather_body, n_chunks, D, row0),
        out_type=jax.ShapeDtypeStruct((B, 2 * D), jnp.float32),
        mesh=mesh,
        scratch_types=[
            pltpu.VMEM((n_chunks, _CHUNK), jnp.int32),
            pltpu.VMEM((n_chunks, _CHUNK), jnp.int32),
            pltpu.VMEM((_CHUNK, D), jnp.float32),
            pltpu.VMEM((_CHUNK, D), jnp.float32),
            pltpu.VMEM((_CHUNK, D), jnp.float32),
            pltpu.VMEM((_CHUNK, D), jnp.float32),
        ] + [pltpu.SemaphoreType.DMA] * 8,
    )
    return k(user.reshape(-1, _CHUNK), item.reshape(-1, _CHUNK),
             user_table, item_table)


def _mlp_body(x_ref, w1_ref, b1_ref, w2_ref, b2_ref,
              w3_ref, b3_ref, wp_ref, bp_ref, out_ref):
    f32, bf16 = jnp.float32, jnp.bfloat16
    h = jnp.dot(x_ref[...].astype(bf16), w1_ref[...].astype(bf16),
                preferred_element_type=f32)
    h = jnp.maximum(h + b1_ref[...], 0.0).astype(bf16)
    h = jnp.dot(h, w2_ref[...].astype(bf16), preferred_element_type=f32)
    h = jnp.maximum(h + b2_ref[...], 0.0).astype(bf16)
    h = jnp.dot(h, w3_ref[...].astype(bf16), preferred_element_type=f32)
    h = jnp.maximum(h + b3_ref[...], 0.0).astype(bf16)
    logit = jnp.dot(h, wp_ref[...].astype(bf16), preferred_element_type=f32)
    p = jax.nn.sigmoid(logit + bp_ref[...])
    out_ref[...] = p.reshape(1, 1, -1)


def _mlp(x_emb, W1, b1, W2, b2, W3, b3, Wp, bp, blk, interpret=False):
    B, D2 = x_emb.shape
    H1 = W1.shape[1]
    H2 = W2.shape[1]
    H3 = W3.shape[1]
    nb = B // blk
    const = lambda shape: pl.BlockSpec(shape, lambda b: (0,) * len(shape))
    out = pl.pallas_call(
        _mlp_body,
        grid=(nb,),
        in_specs=[
            pl.BlockSpec((blk, D2), lambda b: (b, 0)),
            const((D2, H1)),
            const((1, H1)),
            const((H1, H2)),
            const((1, H2)),
            const((H2, H3)),
            const((1, H3)),
            const((H3, 1)),
            const((1, 1)),
        ],
        out_specs=pl.BlockSpec((1, 1, blk), lambda b: (b, 0, 0)),
        out_shape=jax.ShapeDtypeStruct((nb, 1, blk), jnp.float32),
        interpret=interpret,
    )(x_emb, W1, b1, W2, b2, W3, b3, Wp, bp)
    return out.reshape(B)


_N_PIPE = 4


def kernel(user, item, user_table, item_table, W1, b1, W2, b2, W3, b3, Wp, bp):
    B = user.shape[0]
    user = user.astype(jnp.int32)
    item = item.astype(jnp.int32)
    embs = [_gather(user, item, user_table, item_table, c, _N_PIPE)
            for c in range(_N_PIPE)]
    outs = [_mlp(x_e, W1, b1.reshape(1, -1), W2, b2.reshape(1, -1),
                 W3, b3.reshape(1, -1), Wp, bp.reshape(1, 1), blk=2048)
            for x_e in embs]
    return jnp.concatenate(outs)


# speedup vs baseline: 1.0327x; 1.0327x over previous
import functools

import jax
import jax.numpy as jnp
from jax import lax
from jax.experimental import pallas as pl
from jax.experimental.pallas import tpu as pltpu
from jax.experimental.pallas import tpu_sc as plsc

_NC = 2
_NS = 16
_NW = _NC * _NS

_CHUNK = 128


def _gather_body(n_chunks, D, row0, u_idx, i_idx, utab, itab, out,
                 uidx_v, iidx_v, ubuf0, ubuf1, ibuf0, ibuf1,
                 ugs0, ugs1, igs0, igs1, uss0, uss1, iss0, iss1):
    wid = lax.axis_index("s") * _NC + lax.axis_index("c")
    base = wid * (n_chunks * _CHUNK)
    ubuf, ibuf = (ubuf0, ubuf1), (ibuf0, ibuf1)
    ugs, igs = (ugs0, ugs1), (igs0, igs1)
    uss, iss = (uss0, uss1), (iss0, iss1)
    pltpu.sync_copy(u_idx.at[pl.ds(row0 + wid * n_chunks, n_chunks)], uidx_v)
    pltpu.sync_copy(i_idx.at[pl.ds(row0 + wid * n_chunks, n_chunks)], iidx_v)
    ug = [None] * n_chunks
    ig = [None] * n_chunks
    ust = [None] * n_chunks
    ist = [None] * n_chunks
    for c in range(min(2, n_chunks)):
        ug[c] = pltpu.async_copy(utab.at[uidx_v.at[c]], ubuf[c % 2], ugs[c % 2])
        ig[c] = pltpu.async_copy(itab.at[iidx_v.at[c]], ibuf[c % 2], igs[c % 2])
    for c in range(n_chunks):
        s = c % 2
        off = base + c * _CHUNK
        ug[c].wait()
        ust[c] = pltpu.async_copy(
            ubuf[s], out.at[pl.ds(off, _CHUNK), pl.ds(0, D)], uss[s])
        ig[c].wait()
        ist[c] = pltpu.async_copy(
            ibuf[s], out.at[pl.ds(off, _CHUNK), pl.ds(D, D)], iss[s])
        if c + 2 < n_chunks:
            ust[c].wait()
            ug[c + 2] = pltpu.async_copy(utab.at[uidx_v.at[c + 2]], ubuf[s], ugs[s])
            ist[c].wait()
            ig[c + 2] = pltpu.async_copy(itab.at[iidx_v.at[c + 2]], ibuf[s], igs[s])
    for c in range(max(0, n_chunks - 2), n_chunks):
        ust[c].wait()
        ist[c].wait()


@functools.partial(jax.jit, static_argnums=(4, 5))
def _gather(user, item, user_table, item_table, chunk, n_pipe):
    B = user.shape[0] // n_pipe
    D = user_table.shape[1]
    assert B % (_NW * _CHUNK) == 0
    n_chunks = B // (_NW * _CHUNK)
    row0 = chunk * (B // _CHUNK)
    mesh = plsc.VectorSubcoreMesh(core_axis_name="c", subcore_axis_name="s")
    k = pl.kernel(
        functools.partial(_gather_body, n_chunks, D, row0),
        out_type=jax.ShapeDtypeStruct((B, 2 * D), jnp.float32),
        mesh=mesh,
        scratch_types=[
            pltpu.VMEM((n_chunks, _CHUNK), jnp.int32),
            pltpu.VMEM((n_chunks, _CHUNK), jnp.int32),
            pltpu.VMEM((_CHUNK, D), jnp.float32),
            pltpu.VMEM((_CHUNK, D), jnp.float32),
            pltpu.VMEM((_CHUNK, D), jnp.float32),
            pltpu.VMEM((_CHUNK, D), jnp.float32),
        ] + [pltpu.SemaphoreType.DMA] * 8,
    )
    return k(user.reshape(-1, _CHUNK), item.reshape(-1, _CHUNK),
             user_table, item_table)


def _mlp_body(x_ref, w1_ref, b1_ref, w2_ref, b2_ref,
              w3_ref, b3_ref, wp_ref, bp_ref, out_ref):
    f32, bf16 = jnp.float32, jnp.bfloat16
    h = jnp.dot(x_ref[...].astype(bf16), w1_ref[...].astype(bf16),
                preferred_element_type=f32)
    h = jnp.maximum(h + b1_ref[...], 0.0).astype(bf16)
    h = jnp.dot(h, w2_ref[...].astype(bf16), preferred_element_type=f32)
    h = jnp.maximum(h + b2_ref[...], 0.0).astype(bf16)
    h = jnp.dot(h, w3_ref[...].astype(bf16), preferred_element_type=f32)
    h = jnp.maximum(h + b3_ref[...], 0.0).astype(bf16)
    logit = jnp.dot(h, wp_ref[...].astype(bf16), preferred_element_type=f32)
    p = jax.nn.sigmoid(logit + bp_ref[...])
    out_ref[...] = p.reshape(1, 1, -1)


def _mlp(x_emb, W1, b1, W2, b2, W3, b3, Wp, bp, blk, interpret=False):
    B, D2 = x_emb.shape
    H1 = W1.shape[1]
    H2 = W2.shape[1]
    H3 = W3.shape[1]
    nb = B // blk
    const = lambda shape: pl.BlockSpec(shape, lambda b: (0,) * len(shape))
    out = pl.pallas_call(
        _mlp_body,
        grid=(nb,),
        in_specs=[
            pl.BlockSpec((blk, D2), lambda b: (b, 0)),
            const((D2, H1)),
            const((1, H1)),
            const((H1, H2)),
            const((1, H2)),
            const((H2, H3)),
            const((1, H3)),
            const((H3, 1)),
            const((1, 1)),
        ],
        out_specs=pl.BlockSpec((1, 1, blk), lambda b: (b, 0, 0)),
        out_shape=jax.ShapeDtypeStruct((nb, 1, blk), jnp.float32),
        interpret=interpret,
    )(x_emb, W1, b1, W2, b2, W3, b3, Wp, bp)
    return out.reshape(B)


_N_PIPE = 2


def kernel(user, item, user_table, item_table, W1, b1, W2, b2, W3, b3, Wp, bp):
    B = user.shape[0]
    user = user.astype(jnp.int32)
    item = item.astype(jnp.int32)
    embs = [_gather(user, item, user_table, item_table, c, _N_PIPE)
            for c in range(_N_PIPE)]
    outs = [_mlp(x_e, W1, b1.reshape(1, -1), W2, b2.reshape(1, -1),
                 W3, b3.reshape(1, -1), Wp, bp.reshape(1, 1), blk=2048)
            for x_e in embs]
    return jnp.concatenate(outs)


# speedup vs baseline: 1.1917x; 1.1539x over previous
import functools

import jax
import jax.numpy as jnp
from jax import lax
from jax.experimental import pallas as pl
from jax.experimental.pallas import tpu as pltpu
from jax.experimental.pallas import tpu_sc as plsc

_NC = 2
_NS = 16
_NW = _NC * _NS

_CHUNK = 128


def _gather_body(n_chunks, D, row0, u_idx, i_idx, utab, itab, out,
                 uidx_v, iidx_v, ubuf0, ubuf1, ibuf0, ibuf1,
                 ugs0, ugs1, igs0, igs1, uss0, uss1, iss0, iss1):
    wid = lax.axis_index("s") * _NC + lax.axis_index("c")
    base = wid * (n_chunks * _CHUNK)
    ubuf, ibuf = (ubuf0, ubuf1), (ibuf0, ibuf1)
    ugs, igs = (ugs0, ugs1), (igs0, igs1)
    uss, iss = (uss0, uss1), (iss0, iss1)
    pltpu.sync_copy(u_idx.at[pl.ds(row0 + wid * n_chunks, n_chunks)], uidx_v)
    pltpu.sync_copy(i_idx.at[pl.ds(row0 + wid * n_chunks, n_chunks)], iidx_v)
    ug = [None] * n_chunks
    ig = [None] * n_chunks
    ust = [None] * n_chunks
    ist = [None] * n_chunks
    for c in range(min(2, n_chunks)):
        ug[c] = pltpu.async_copy(utab.at[uidx_v.at[c]], ubuf[c % 2], ugs[c % 2])
        ig[c] = pltpu.async_copy(itab.at[iidx_v.at[c]], ibuf[c % 2], igs[c % 2])
    for c in range(n_chunks):
        s = c % 2
        off = base + c * _CHUNK
        ug[c].wait()
        ust[c] = pltpu.async_copy(
            ubuf[s], out.at[pl.ds(off, _CHUNK), pl.ds(0, D)], uss[s])
        ig[c].wait()
        ist[c] = pltpu.async_copy(
            ibuf[s], out.at[pl.ds(off, _CHUNK), pl.ds(D, D)], iss[s])
        if c + 2 < n_chunks:
            ust[c].wait()
            ug[c + 2] = pltpu.async_copy(utab.at[uidx_v.at[c + 2]], ubuf[s], ugs[s])
            ist[c].wait()
            ig[c + 2] = pltpu.async_copy(itab.at[iidx_v.at[c + 2]], ibuf[s], igs[s])
    for c in range(max(0, n_chunks - 2), n_chunks):
        ust[c].wait()
        ist[c].wait()


@functools.partial(jax.jit, static_argnums=(4, 5))
def _gather(user, item, user_table, item_table, chunk, n_pipe):
    B = user.shape[0] // n_pipe
    D = user_table.shape[1]
    assert B % (_NW * _CHUNK) == 0
    n_chunks = B // (_NW * _CHUNK)
    row0 = chunk * (B // _CHUNK)
    mesh = plsc.VectorSubcoreMesh(core_axis_name="c", subcore_axis_name="s")
    k = pl.kernel(
        functools.partial(_gather_body, n_chunks, D, row0),
        out_type=jax.ShapeDtypeStruct((B, 2 * D), jnp.float32),
        mesh=mesh,
        scratch_types=[
            pltpu.VMEM((n_chunks, _CHUNK), jnp.int32),
            pltpu.VMEM((n_chunks, _CHUNK), jnp.int32),
            pltpu.VMEM((_CHUNK, D), jnp.float32),
            pltpu.VMEM((_CHUNK, D), jnp.float32),
            pltpu.VMEM((_CHUNK, D), jnp.float32),
            pltpu.VMEM((_CHUNK, D), jnp.float32),
        ] + [pltpu.SemaphoreType.DMA] * 8,
    )
    return k(user.reshape(-1, _CHUNK), item.reshape(-1, _CHUNK),
             user_table, item_table)


def _mlp_body(x_ref, w1_ref, b1_ref, w2_ref, b2_ref,
              w3_ref, b3_ref, wp_ref, bp_ref, out_ref):
    f32, bf16 = jnp.float32, jnp.bfloat16
    h = lax.dot_general(w1_ref[...].astype(bf16), x_ref[...].astype(bf16),
                        dimension_numbers=(((0,), (1,)), ((), ())),
                        preferred_element_type=f32)
    h = jnp.maximum(h + b1_ref[...], 0.0).astype(bf16)
    h = lax.dot_general(w2_ref[...].astype(bf16), h,
                        dimension_numbers=(((0,), (0,)), ((), ())),
                        preferred_element_type=f32)
    h = jnp.maximum(h + b2_ref[...], 0.0).astype(bf16)
    h = lax.dot_general(w3_ref[...].astype(bf16), h,
                        dimension_numbers=(((0,), (0,)), ((), ())),
                        preferred_element_type=f32)
    h = jnp.maximum(h + b3_ref[...], 0.0).astype(bf16)
    logit = lax.dot_general(wp_ref[...].astype(bf16), h,
                            dimension_numbers=(((0,), (0,)), ((), ())),
                            preferred_element_type=f32)
    out_ref[...] = jax.nn.sigmoid(logit + bp_ref[...]).reshape(1, 1, -1)


def _mlp(x_emb, W1, b1, W2, b2, W3, b3, Wp, bp, blk, interpret=False):
    B, D2 = x_emb.shape
    H1 = W1.shape[1]
    H2 = W2.shape[1]
    H3 = W3.shape[1]
    nb = B // blk
    const = lambda shape: pl.BlockSpec(shape, lambda b: (0,) * len(shape))
    out = pl.pallas_call(
        _mlp_body,
        grid=(nb,),
        in_specs=[
            pl.BlockSpec((blk, D2), lambda b: (b, 0)),
            const((D2, H1)),
            const((H1, 1)),
            const((H1, H2)),
            const((H2, 1)),
            const((H2, H3)),
            const((H3, 1)),
            const((H3, 1)),
            const((1, 1)),
        ],
        out_specs=pl.BlockSpec((1, 1, blk), lambda b: (b, 0, 0)),
        out_shape=jax.ShapeDtypeStruct((nb, 1, blk), jnp.float32),
        interpret=interpret,
    )(x_emb, W1, b1, W2, b2, W3, b3, Wp, bp)
    return out.reshape(B)


_N_PIPE = 2


def kernel(user, item, user_table, item_table, W1, b1, W2, b2, W3, b3, Wp, bp):
    B = user.shape[0]
    user = user.astype(jnp.int32)
    item = item.astype(jnp.int32)
    embs = [_gather(user, item, user_table, item_table, c, _N_PIPE)
            for c in range(_N_PIPE)]
    outs = [_mlp(x_e, W1, b1.reshape(-1, 1), W2, b2.reshape(-1, 1),
                 W3, b3.reshape(-1, 1), Wp, bp.reshape(1, 1), blk=2048)
            for x_e in embs]
    return jnp.concatenate(outs)


# speedup vs baseline: 1.2126x; 1.0176x over previous
import functools

import jax
import jax.numpy as jnp
from jax import lax
from jax.experimental import pallas as pl
from jax.experimental.pallas import tpu as pltpu
from jax.experimental.pallas import tpu_sc as plsc

_NC = 2
_NS = 16
_NW = _NC * _NS

_CHUNK = 128


def _gather_body(n_chunks, D, row0, u_idx, i_idx, utab, itab, out,
                 uidx_v, iidx_v, ubuf0, ubuf1, ibuf0, ibuf1,
                 ugs0, ugs1, igs0, igs1, uss0, uss1, iss0, iss1):
    wid = lax.axis_index("s") * _NC + lax.axis_index("c")
    base = wid * (n_chunks * _CHUNK)
    ubuf, ibuf = (ubuf0, ubuf1), (ibuf0, ibuf1)
    ugs, igs = (ugs0, ugs1), (igs0, igs1)
    uss, iss = (uss0, uss1), (iss0, iss1)
    pltpu.sync_copy(u_idx.at[pl.ds(row0 + wid * n_chunks, n_chunks)], uidx_v)
    pltpu.sync_copy(i_idx.at[pl.ds(row0 + wid * n_chunks, n_chunks)], iidx_v)
    ug = [None] * n_chunks
    ig = [None] * n_chunks
    ust = [None] * n_chunks
    ist = [None] * n_chunks
    for c in range(min(2, n_chunks)):
        ug[c] = pltpu.async_copy(utab.at[uidx_v.at[c]], ubuf[c % 2], ugs[c % 2])
        ig[c] = pltpu.async_copy(itab.at[iidx_v.at[c]], ibuf[c % 2], igs[c % 2])
    for c in range(n_chunks):
        s = c % 2
        off = base + c * _CHUNK
        ug[c].wait()
        ust[c] = pltpu.async_copy(
            ubuf[s], out.at[pl.ds(off, _CHUNK), pl.ds(0, D)], uss[s])
        ig[c].wait()
        ist[c] = pltpu.async_copy(
            ibuf[s], out.at[pl.ds(off, _CHUNK), pl.ds(D, D)], iss[s])
        if c + 2 < n_chunks:
            ust[c].wait()
            ug[c + 2] = pltpu.async_copy(utab.at[uidx_v.at[c + 2]], ubuf[s], ugs[s])
            ist[c].wait()
            ig[c + 2] = pltpu.async_copy(itab.at[iidx_v.at[c + 2]], ibuf[s], igs[s])
    for c in range(max(0, n_chunks - 2), n_chunks):
        ust[c].wait()
        ist[c].wait()


@functools.partial(jax.jit, static_argnums=(4, 5))
def _gather(user, item, user_table, item_table, chunk, n_pipe):
    B = user.shape[0] // n_pipe
    D = user_table.shape[1]
    assert B % (_NW * _CHUNK) == 0
    n_chunks = B // (_NW * _CHUNK)
    row0 = chunk * (B // _CHUNK)
    mesh = plsc.VectorSubcoreMesh(core_axis_name="c", subcore_axis_name="s")
    k = pl.kernel(
        functools.partial(_gather_body, n_chunks, D, row0),
        out_type=jax.ShapeDtypeStruct((B, 2 * D), jnp.float32),
        mesh=mesh,
        scratch_types=[
            pltpu.VMEM((n_chunks, _CHUNK), jnp.int32),
            pltpu.VMEM((n_chunks, _CHUNK), jnp.int32),
            pltpu.VMEM((_CHUNK, D), jnp.float32),
            pltpu.VMEM((_CHUNK, D), jnp.float32),
            pltpu.VMEM((_CHUNK, D), jnp.float32),
            pltpu.VMEM((_CHUNK, D), jnp.float32),
        ] + [pltpu.SemaphoreType.DMA] * 8,
    )
    return k(user.reshape(-1, _CHUNK), item.reshape(-1, _CHUNK),
             user_table, item_table)


def _mlp_body(x_ref, w1_ref, b1_ref, w2_ref, b2_ref,
              w3_ref, b3_ref, wp_ref, bp_ref, out_ref):
    f32, bf16 = jnp.float32, jnp.bfloat16
    h = lax.dot_general(w1_ref[...].astype(bf16), x_ref[...].astype(bf16),
                        dimension_numbers=(((0,), (1,)), ((), ())),
                        preferred_element_type=f32)
    h = jnp.maximum(h + b1_ref[...], 0.0).astype(bf16)
    h = lax.dot_general(w2_ref[...].astype(bf16), h,
                        dimension_numbers=(((0,), (0,)), ((), ())),
                        preferred_element_type=f32)
    h = jnp.maximum(h + b2_ref[...], 0.0).astype(bf16)
    h = lax.dot_general(w3_ref[...].astype(bf16), h,
                        dimension_numbers=(((0,), (0,)), ((), ())),
                        preferred_element_type=f32)
    h = jnp.maximum(h + b3_ref[...], 0.0).astype(bf16)
    logit = lax.dot_general(wp_ref[...].astype(bf16), h,
                            dimension_numbers=(((0,), (0,)), ((), ())),
                            preferred_element_type=f32)
    out_ref[...] = jax.nn.sigmoid(logit + bp_ref[...]).reshape(1, 1, -1)


def _mlp(x_emb, W1, b1, W2, b2, W3, b3, Wp, bp, blk, interpret=False):
    B, D2 = x_emb.shape
    H1 = W1.shape[1]
    H2 = W2.shape[1]
    H3 = W3.shape[1]
    nb = B // blk
    const = lambda shape: pl.BlockSpec(shape, lambda b: (0,) * len(shape))
    out = pl.pallas_call(
        _mlp_body,
        grid=(nb,),
        in_specs=[
            pl.BlockSpec((blk, D2), lambda b: (b, 0)),
            const((D2, H1)),
            const((H1, 1)),
            const((H1, H2)),
            const((H2, 1)),
            const((H2, H3)),
            const((H3, 1)),
            const((H3, 1)),
            const((1, 1)),
        ],
        out_specs=pl.BlockSpec((1, 1, blk), lambda b: (b, 0, 0)),
        out_shape=jax.ShapeDtypeStruct((nb, 1, blk), jnp.float32),
        interpret=interpret,
    )(x_emb, W1, b1, W2, b2, W3, b3, Wp, bp)
    return out.reshape(B)


_N_PIPE = 2


def kernel(user, item, user_table, item_table, W1, b1, W2, b2, W3, b3, Wp, bp):
    B = user.shape[0]
    user = user.astype(jnp.int32)
    item = item.astype(jnp.int32)
    embs = [_gather(user, item, user_table, item_table, c, _N_PIPE)
            for c in range(_N_PIPE)]
    outs = [_mlp(x_e, W1, b1.reshape(-1, 1), W2, b2.reshape(-1, 1),
                 W3, b3.reshape(-1, 1), Wp, bp.reshape(1, 1), blk=4096)
            for x_e in embs]
    return jnp.concatenate(outs)
